# Initial kernel scaffold; baseline (speedup 1.0000x reference)
import jax
import jax.numpy as jnp
from jax.experimental import pallas as pl


def kernel(x, edge_index, batch, Wg1, asrc1, adst1, bg1, Wf1, bf1, Wg2, asrc2, adst2, bg2, Wf2, bf2, Wg3, asrc3, adst3, bg3, Wf3, bf3, Wm1, bm1, Wm2, bm2):
    raise NotImplementedError("write your pallas kernel here")


# speedup vs baseline: 52.2805x; 52.2805x over previous
import functools

import jax
import jax.numpy as jnp
from jax import lax
from jax.experimental import pallas as pl
from jax.experimental.pallas import tpu as pltpu
from jax.experimental.pallas import tpu_sc as plsc

NF = 15
N = 50000
G = 512

NPAD = 50176
GP = 520
NW = 32
CHUNK = 1024
RPT = NPAD // 16

_E2 = 1600000 + N
_NCK = -(-_E2 // (NW * CHUNK))
EPW = _NCK * CHUNK
EPAD = NW * EPW

_f32 = jnp.float32
_i32 = jnp.int32
_PREC = jax.lax.Precision.HIGHEST


def _tc_pre_body(x_ref, w_ref, asrc_ref, adst_ref, g16_ref, ad_ref):
    h = jax.lax.dot_general(x_ref[...], w_ref[...],
                            (((1,), (1,)), ((), ())),
                            preferred_element_type=_f32, precision=_PREC)
    a_s = jnp.dot(h, asrc_ref[...], preferred_element_type=_f32,
                  precision=_PREC)
    g16_ref[...] = jnp.concatenate([h, a_s], axis=1)
    ad_ref[...] = jnp.dot(h, adst_ref[...], preferred_element_type=_f32,
                          precision=_PREC)


BLK = NPAD // 16
_GRID = NPAD // BLK


def _bspec(shape):
    nd = len(shape)
    return pl.BlockSpec(shape, lambda i, _n=nd: (0,) * _n)


def _rspec(width, lead=None):
    if lead is None:
        return pl.BlockSpec((BLK, width), lambda i: (i, 0))
    return pl.BlockSpec((lead, BLK, width), lambda i: (0, i, 0))


def _tc_pre(x_p, wg, asrc, adst):
    return pl.pallas_call(
        _tc_pre_body,
        grid=(_GRID,),
        in_specs=[_rspec(NF), _bspec((NF, NF)), _bspec((NF, 1)),
                  _bspec((NF, 1))],
        out_specs=(_rspec(16), _rspec(1)),
        out_shape=(jax.ShapeDtypeStruct((NPAD, 16), _f32),
                   jax.ShapeDtypeStruct((NPAD, 1), _f32)),
    )(x_p, wg, asrc, adst)


def _lanes15(col):
    return jax.lax.dot_general(col, jnp.ones((1, NF), _f32),
                               (((1,), (0,)), ((), ())),
                               preferred_element_type=_f32,
                               precision=_PREC)


def _combine_gat(acc_ref, bg_ref):
    a = acc_ref[0] + acc_ref[1]
    rden = 1.0 / jnp.maximum(a[:, 15:16], 1e-30)
    return a[:, :15] * _lanes15(rden) + bg_ref[...]


def _tc_mid_body(acc_ref, bg_ref, wf_ref, bf_ref, wg_ref, asrc_ref,
                 adst_ref, g16_ref, ad_ref):
    gat = _combine_gat(acc_ref, bg_ref)
    hf = jax.nn.relu(
        jax.lax.dot_general(gat, wf_ref[...], (((1,), (1,)), ((), ())),
                            preferred_element_type=_f32, precision=_PREC)
        + bf_ref[...])
    h = jax.lax.dot_general(hf, wg_ref[...], (((1,), (1,)), ((), ())),
                            preferred_element_type=_f32, precision=_PREC)
    a_s = jnp.dot(h, asrc_ref[...], preferred_element_type=_f32,
                  precision=_PREC)
    g16_ref[...] = jnp.concatenate([h, a_s], axis=1)
    ad_ref[...] = jnp.dot(h, adst_ref[...], preferred_element_type=_f32,
                          precision=_PREC)


def _tc_mid(acc, bg, wf, bf, wg, asrc, adst):
    return pl.pallas_call(
        _tc_mid_body,
        grid=(_GRID,),
        in_specs=[_rspec(16, lead=2), _bspec((1, NF)), _bspec((NF, NF)),
                  _bspec((1, NF)), _bspec((NF, NF)), _bspec((NF, 1)),
                  _bspec((NF, 1))],
        out_specs=(_rspec(16), _rspec(1)),
        out_shape=(jax.ShapeDtypeStruct((NPAD, 16), _f32),
                   jax.ShapeDtypeStruct((NPAD, 1), _f32)),
    )(acc, bg, wf, bf, wg, asrc, adst)


def _tc_post_body(acc_ref, bg_ref, wf_ref, bf_ref, hfin_ref):
    gat = _combine_gat(acc_ref, bg_ref)
    h = jax.lax.dot_general(gat, wf_ref[...], (((1,), (1,)), ((), ())),
                            preferred_element_type=_f32, precision=_PREC) \
        + bf_ref[...]
    hfin_ref[...] = jnp.concatenate(
        [h, jnp.ones((BLK, 1), _f32)], axis=1)


def _tc_post(acc, bg, wf, bf):
    return pl.pallas_call(
        _tc_post_body,
        grid=(_GRID,),
        in_specs=[_rspec(16, lead=2), _bspec((1, NF)), _bspec((NF, NF)),
                  _bspec((1, NF))],
        out_specs=_rspec(16),
        out_shape=jax.ShapeDtypeStruct((NPAD, 16), _f32),
    )(acc, bg, wf, bf)


def _tc_head_body(psum_ref, pmax_ref, wm1_ref, bm1_ref, wm2_ref, bm2_ref,
                  out_ref):
    ssum = jnp.sum(psum_ref[...], axis=0)[:G]
    smaxr = jnp.max(pmax_ref[...], axis=0)[:G]
    cnt = ssum[:, 15:16]
    smax = jnp.where(_lanes15(cnt) > 0, smaxr[:, :15], 0.0)
    smean = ssum[:, :15] * _lanes15(1.0 / jnp.maximum(cnt, 1.0))
    z = jnp.concatenate([ssum[:, :15], smax, smean], axis=1)
    z = jax.nn.relu(
        jax.lax.dot_general(z, wm1_ref[...], (((1,), (1,)), ((), ())),
                            preferred_element_type=_f32, precision=_PREC)
        + bm1_ref[...])
    z = jax.lax.dot_general(z, wm2_ref[...], (((1,), (1,)), ((), ())),
                            preferred_element_type=_f32, precision=_PREC) \
        + bm2_ref[...]
    out_ref[...] = jax.nn.sigmoid(z)[:, :1]


def _tc_head(psum, pmax, wm1, bm1, wm2, bm2):
    return pl.pallas_call(
        _tc_head_body,
        out_shape=jax.ShapeDtypeStruct((G, 1), _f32),
    )(psum, pmax, wm1, bm1, wm2, bm2)


def _edge_body(g16_hbm, ad_hbm, src_hbm, dst_hbm, out_hbm,
               accs, ads, srcb, dstb, adch, rows, sem, sem2):
    cid = lax.axis_index("c")
    sid = lax.axis_index("s")
    wid = cid * 16 + sid
    iota = lax.iota(_i32, 16)
    zeros16 = jnp.zeros((16,), _f32)

    def _zrow(i, _):
        plsc.store_scatter(rows, [jnp.broadcast_to(i, (16,)), iota], zeros16)
        return _
    lax.fori_loop(0, CHUNK, _zrow, None)
    abase = sid * RPT
    for j in range(RPT // CHUNK):
        pltpu.sync_copy(rows, accs.at[pl.ds(abase + j * CHUNK, CHUNK)])
    rem = RPT % CHUNK
    if rem:
        pltpu.sync_copy(rows.at[pl.ds(0, rem)],
                        accs.at[pl.ds(abase + (RPT // CHUNK) * CHUNK, rem)])

    pltpu.sync_copy(ad_hbm.at[pl.ds(sid * RPT, RPT)],
                    ads.at[pl.ds(sid * RPT, RPT)])
    plsc.subcore_barrier()

    ebase = wid * EPW

    def _chunk(k, _):
        eb = ebase + k * CHUNK
        pltpu.sync_copy(src_hbm.at[pl.ds(eb, CHUNK)], srcb)
        pltpu.sync_copy(dst_hbm.at[pl.ds(eb, CHUNK)], dstb)
        gcp = pltpu.async_copy(g16_hbm.at[srcb], rows, sem)
        acp = pltpu.async_copy(ads.at[dstb], adch, sem2)
        gcp.wait()
        acp.wait()

        def _grp(j, _):
            rid = iota + j * 16
            adv = plsc.load_gather(adch, [rid])
            lane15 = jnp.broadcast_to(15, (16,))
            asv = plsc.load_gather(rows, [rid, lane15])
            e = asv + adv
            w = jnp.exp(jnp.where(e >= 0, e, 0.2 * e))
            plsc.store_scatter(rows, [rid, lane15], w)
            for f in range(15):
                lane = jnp.broadcast_to(f, (16,))
                col = plsc.load_gather(rows, [rid, lane])
                plsc.store_scatter(rows, [rid, lane], col * w)
            return _
        lax.fori_loop(0, CHUNK // 16, _grp, None)
        pltpu.sync_copy(rows, accs.at[dstb], add=True)
        return _
    lax.fori_loop(0, EPW // CHUNK, _chunk, None)

    plsc.subcore_barrier()
    pltpu.sync_copy(accs.at[pl.ds(abase, RPT)],
                    out_hbm.at[cid, pl.ds(abase, RPT)])


def _sc_edge(g16, ad, src_p, dst_p):
    mesh = plsc.VectorSubcoreMesh(core_axis_name="c", subcore_axis_name="s")
    f = pl.kernel(
        _edge_body,
        out_type=jax.ShapeDtypeStruct((2, NPAD, 16), _f32),
        mesh=mesh,
        compiler_params=pltpu.CompilerParams(needs_layout_passes=False,
                                             use_tc_tiling_on_sc=False),
        scratch_types=[
            pltpu.VMEM_SHARED((NPAD, 16), _f32),
            pltpu.VMEM_SHARED((NPAD,), _f32),
            pltpu.VMEM((CHUNK,), _i32),
            pltpu.VMEM((CHUNK,), _i32),
            pltpu.VMEM((CHUNK,), _f32),
            pltpu.VMEM((CHUNK, 16), _f32),
            pltpu.SemaphoreType.DMA,
            pltpu.SemaphoreType.DMA,
        ],
    )
    return f(g16, ad, src_p, dst_p)


NPW = NPAD // NW


PC = NPW // 2


def _pool_body(hfin_hbm, batch_hbm, psum_hbm, pmax_hbm,
               rowsb, batchb, sacc, macc, sem):
    cid = lax.axis_index("c")
    sid = lax.axis_index("s")
    wid = cid * 16 + sid
    iota = lax.iota(_i32, 16)
    zeros16 = jnp.zeros((16,), _f32)
    neg16 = jnp.full((16,), -3.4e38, _f32)

    def _init(i, _):
        idx = jnp.broadcast_to(i, (16,))
        plsc.store_scatter(sacc, [idx, iota], zeros16)
        plsc.store_scatter(macc, [idx, iota], neg16)
        return _
    lax.fori_loop(0, GP, _init, None)

    def _chunk(k, _):
        nb = wid * NPW + k * PC
        pltpu.sync_copy(hfin_hbm.at[pl.ds(nb, PC)], rowsb)
        pltpu.sync_copy(batch_hbm.at[pl.ds(nb, PC)], batchb)

        def _node(i, _):
            ridx = jnp.broadcast_to(i, (16,))
            bidx = plsc.load_gather(batchb, [ridx])
            row = plsc.load_gather(rowsb, [ridx, iota])
            s_old = plsc.load_gather(sacc, [bidx, iota])
            plsc.store_scatter(sacc, [bidx, iota], s_old + row)
            m_old = plsc.load_gather(macc, [bidx, iota])
            plsc.store_scatter(macc, [bidx, iota], jnp.maximum(m_old, row))
            return _
        lax.fori_loop(0, PC, _node, None)
        return _
    lax.fori_loop(0, NPW // PC, _chunk, None)

    pltpu.sync_copy(sacc, psum_hbm.at[wid])
    pltpu.sync_copy(macc, pmax_hbm.at[wid])


def _sc_pool(hfin, batch_p):
    mesh = plsc.VectorSubcoreMesh(core_axis_name="c", subcore_axis_name="s")
    f = pl.kernel(
        _pool_body,
        out_type=(jax.ShapeDtypeStruct((NW, GP, 16), _f32),
                  jax.ShapeDtypeStruct((NW, GP, 16), _f32)),
        mesh=mesh,
        compiler_params=pltpu.CompilerParams(needs_layout_passes=False,
                                             use_tc_tiling_on_sc=False),
        scratch_types=[
            pltpu.VMEM((PC, 16), _f32),
            pltpu.VMEM((PC,), _i32),
            pltpu.VMEM((GP, 16), _f32),
            pltpu.VMEM((GP, 16), _f32),
            pltpu.SemaphoreType.DMA,
        ],
    )
    return f(hfin, batch_p)


def kernel(x, edge_index, batch, Wg1, asrc1, adst1, bg1, Wf1, bf1,
           Wg2, asrc2, adst2, bg2, Wf2, bf2,
           Wg3, asrc3, adst3, bg3, Wf3, bf3,
           Wm1, bm1, Wm2, bm2):
    loops = jnp.arange(N, dtype=_i32)
    epad = jnp.full((EPAD - _E2,), N, _i32)
    src_p = jnp.concatenate([edge_index[0], loops, epad])
    dst_p = jnp.concatenate([edge_index[1], loops, epad])
    x_p = jnp.pad(x, ((0, NPAD - N), (0, 0)))
    batch_p = jnp.pad(batch, (0, NPAD - N), constant_values=G)

    g16, ad = _tc_pre(x_p, Wg1, asrc1.reshape(NF, 1), adst1.reshape(NF, 1))
    acc = _sc_edge(g16, ad.reshape(NPAD), src_p, dst_p)
    g16, ad = _tc_mid(acc, bg1.reshape(1, NF), Wf1, bf1.reshape(1, NF),
                      Wg2, asrc2.reshape(NF, 1), adst2.reshape(NF, 1))
    acc = _sc_edge(g16, ad.reshape(NPAD), src_p, dst_p)
    g16, ad = _tc_mid(acc, bg2.reshape(1, NF), Wf2, bf2.reshape(1, NF),
                      Wg3, asrc3.reshape(NF, 1), adst3.reshape(NF, 1))
    acc = _sc_edge(g16, ad.reshape(NPAD), src_p, dst_p)
    hfin = _tc_post(acc, bg3.reshape(1, NF), Wf3, bf3.reshape(1, NF))
    psum, pmax = _sc_pool(hfin, batch_p)
    wm2_p = jnp.pad(Wm2, ((0, 7), (0, 0)))
    bm2_p = jnp.pad(bm2.reshape(1, 1), ((0, 0), (0, 7)))
    return _tc_head(psum, pmax, Wm1, bm1.reshape(1, 3 * NF), wm2_p, bm2_p)


# speedup vs baseline: 78.2040x; 1.4959x over previous
import functools

import jax
import jax.numpy as jnp
from jax import lax
from jax.experimental import pallas as pl
from jax.experimental.pallas import tpu as pltpu
from jax.experimental.pallas import tpu_sc as plsc

NF = 15
N = 50000
G = 512

NPAD = 50176
GP = 520
NW = 32
CHUNK = 480
RPT = NPAD // 16

_E2 = 1600000 + N
_NCK = -(-_E2 // (NW * CHUNK * 6)) * 6
EPW = _NCK * CHUNK
EPAD = NW * EPW
_NT = _NCK // 6

_f32 = jnp.float32
_i32 = jnp.int32
_PREC = jax.lax.Precision.HIGHEST


def _tc_pre_body(x_ref, w_ref, asrc_ref, adst_ref, g16_ref, ad_ref):
    h = jax.lax.dot_general(x_ref[...], w_ref[...],
                            (((1,), (1,)), ((), ())),
                            preferred_element_type=_f32, precision=_PREC)
    a_s = jnp.dot(h, asrc_ref[...], preferred_element_type=_f32,
                  precision=_PREC)
    g16_ref[...] = jnp.concatenate([h, a_s], axis=1)
    ad_ref[...] = jnp.dot(h, adst_ref[...], preferred_element_type=_f32,
                          precision=_PREC)


BLK = NPAD // 16
_GRID = NPAD // BLK


def _bspec(shape):
    nd = len(shape)
    return pl.BlockSpec(shape, lambda i, _n=nd: (0,) * _n)


def _rspec(width, lead=None):
    if lead is None:
        return pl.BlockSpec((BLK, width), lambda i: (i, 0))
    return pl.BlockSpec((lead, BLK, width), lambda i: (0, i, 0))


def _tc_pre(x_p, wg, asrc, adst):
    return pl.pallas_call(
        _tc_pre_body,
        grid=(_GRID,),
        in_specs=[_rspec(NF), _bspec((NF, NF)), _bspec((NF, 1)),
                  _bspec((NF, 1))],
        out_specs=(_rspec(16), _rspec(1)),
        out_shape=(jax.ShapeDtypeStruct((NPAD, 16), _f32),
                   jax.ShapeDtypeStruct((NPAD, 1), _f32)),
    )(x_p, wg, asrc, adst)


def _lanes15(col):
    return jax.lax.dot_general(col, jnp.ones((1, NF), _f32),
                               (((1,), (0,)), ((), ())),
                               preferred_element_type=_f32,
                               precision=_PREC)


def _combine_gat(acc_ref, bg_ref):
    a = acc_ref[0] + acc_ref[1]
    rden = 1.0 / jnp.maximum(a[:, 15:16], 1e-30)
    return a[:, :15] * _lanes15(rden) + bg_ref[...]


def _tc_mid_body(acc_ref, bg_ref, wf_ref, bf_ref, wg_ref, asrc_ref,
                 adst_ref, g16_ref, ad_ref):
    gat = _combine_gat(acc_ref, bg_ref)
    hf = jax.nn.relu(
        jax.lax.dot_general(gat, wf_ref[...], (((1,), (1,)), ((), ())),
                            preferred_element_type=_f32, precision=_PREC)
        + bf_ref[...])
    h = jax.lax.dot_general(hf, wg_ref[...], (((1,), (1,)), ((), ())),
                            preferred_element_type=_f32, precision=_PREC)
    a_s = jnp.dot(h, asrc_ref[...], preferred_element_type=_f32,
                  precision=_PREC)
    g16_ref[...] = jnp.concatenate([h, a_s], axis=1)
    ad_ref[...] = jnp.dot(h, adst_ref[...], preferred_element_type=_f32,
                          precision=_PREC)


def _tc_mid(acc, bg, wf, bf, wg, asrc, adst):
    return pl.pallas_call(
        _tc_mid_body,
        grid=(_GRID,),
        in_specs=[_rspec(16, lead=2), _bspec((1, NF)), _bspec((NF, NF)),
                  _bspec((1, NF)), _bspec((NF, NF)), _bspec((NF, 1)),
                  _bspec((NF, 1))],
        out_specs=(_rspec(16), _rspec(1)),
        out_shape=(jax.ShapeDtypeStruct((NPAD, 16), _f32),
                   jax.ShapeDtypeStruct((NPAD, 1), _f32)),
    )(acc, bg, wf, bf, wg, asrc, adst)


def _tc_post_body(acc_ref, bg_ref, wf_ref, bf_ref, hfin_ref):
    gat = _combine_gat(acc_ref, bg_ref)
    h = jax.lax.dot_general(gat, wf_ref[...], (((1,), (1,)), ((), ())),
                            preferred_element_type=_f32, precision=_PREC) \
        + bf_ref[...]
    hfin_ref[...] = jnp.concatenate(
        [h, jnp.ones((BLK, 1), _f32)], axis=1)


def _tc_post(acc, bg, wf, bf):
    return pl.pallas_call(
        _tc_post_body,
        grid=(_GRID,),
        in_specs=[_rspec(16, lead=2), _bspec((1, NF)), _bspec((NF, NF)),
                  _bspec((1, NF))],
        out_specs=_rspec(16),
        out_shape=jax.ShapeDtypeStruct((NPAD, 16), _f32),
    )(acc, bg, wf, bf)


def _tc_head_body(psum_ref, pmax_ref, wm1_ref, bm1_ref, wm2_ref, bm2_ref,
                  out_ref):
    ssum = jnp.sum(psum_ref[...], axis=0)[:G]
    smaxr = jnp.max(pmax_ref[...], axis=0)[:G]
    cnt = ssum[:, 15:16]
    smax = jnp.where(_lanes15(cnt) > 0, smaxr[:, :15], 0.0)
    smean = ssum[:, :15] * _lanes15(1.0 / jnp.maximum(cnt, 1.0))
    z = jnp.concatenate([ssum[:, :15], smax, smean], axis=1)
    z = jax.nn.relu(
        jax.lax.dot_general(z, wm1_ref[...], (((1,), (1,)), ((), ())),
                            preferred_element_type=_f32, precision=_PREC)
        + bm1_ref[...])
    z = jax.lax.dot_general(z, wm2_ref[...], (((1,), (1,)), ((), ())),
                            preferred_element_type=_f32, precision=_PREC) \
        + bm2_ref[...]
    out_ref[...] = jax.nn.sigmoid(z)[:, :1]


def _tc_head(psum, pmax, wm1, bm1, wm2, bm2):
    return pl.pallas_call(
        _tc_head_body,
        out_shape=jax.ShapeDtypeStruct((G, 1), _f32),
    )(psum, pmax, wm1, bm1, wm2, bm2)


def _edge_body(g16_hbm, ad_hbm, edges_hbm, out_hbm, accs, ads,
               ebs, rws, ads_ch, ses, srs, sas, sss):
    cid = lax.axis_index("c")
    sid = lax.axis_index("s")
    wid = cid * 16 + sid
    iota = lax.iota(_i32, 16)
    zeros16 = jnp.zeros((16,), _f32)
    rows0 = rws[0]

    def _zrow(i, _):
        plsc.store_scatter(rows0, [jnp.broadcast_to(i, (16,)), iota],
                           zeros16)
        return _
    lax.fori_loop(0, CHUNK, _zrow, None)
    abase = sid * RPT
    for j in range(RPT // CHUNK):
        pltpu.sync_copy(rows0, accs.at[pl.ds(abase + j * CHUNK, CHUNK)])
    rem = RPT % CHUNK
    if rem:
        pltpu.sync_copy(rows0.at[pl.ds(0, rem)],
                        accs.at[pl.ds(abase + (RPT // CHUNK) * CHUNK, rem)])

    pltpu.sync_copy(ad_hbm.at[pl.ds(sid * RPT, RPT)],
                    ads.at[pl.ds(sid * RPT, RPT)])
    plsc.subcore_barrier()

    ebase = wid * EPW

    def _startE(off, c6):
        pltpu.async_copy(edges_hbm.at[:, pl.ds(off, CHUNK)], ebs[c6],
                         ses[c6])

    def _waitE(c6):
        pltpu.make_async_copy(edges_hbm.at[:, pl.ds(0, CHUNK)], ebs[c6],
                              ses[c6]).wait()

    def _startR(c6, c3):
        pltpu.async_copy(g16_hbm.at[ebs[c6].at[0]], rws[c3], srs[c3])
        pltpu.async_copy(ads.at[ebs[c6].at[1]], ads_ch[c3], sas[c3])

    def _waitR(c6, c3):
        pltpu.make_async_copy(g16_hbm.at[ebs[c6].at[0]], rws[c3],
                              srs[c3]).wait()
        pltpu.make_async_copy(ads.at[ebs[c6].at[1]], ads_ch[c3],
                              sas[c3]).wait()

    def _startS(c6, c3):
        pltpu.async_copy(rws[c3], accs.at[ebs[c6].at[1]], sss[c3],
                         add=True)

    def _waitS(c6, c3):
        pltpu.make_async_copy(rws[c3], accs.at[ebs[c6].at[1]],
                              sss[c3]).wait()

    def _compute(c6, c3):
        rows, adch = rws[c3], ads_ch[c3]
        lane15 = jnp.broadcast_to(15, (16,))

        def _grp(j, _):
            rid = iota + j * 16
            adv = plsc.load_gather(adch, [rid])
            asv = plsc.load_gather(rows, [rid, lane15])
            e = asv + adv
            w = jnp.exp(jnp.where(e >= 0, e, 0.2 * e))
            plsc.store_scatter(rows, [rid, lane15], w)
            for f in range(15):
                lane = jnp.broadcast_to(f, (16,))
                col = plsc.load_gather(rows, [rid, lane])
                plsc.store_scatter(rows, [rid, lane], col * w)
            return _
        lax.fori_loop(0, CHUNK // 16, _grp, None)

    pltpu.sync_copy(edges_hbm.at[:, pl.ds(ebase, CHUNK)], ebs[0])
    _startR(0, 0)
    _startE(ebase + CHUNK, 1)

    def _iter(i, _):
        base = ebase + 6 * i * CHUNK
        for c in range(6):
            c6, c3 = c, c % 3
            n6, n3 = (c + 1) % 6, (c + 1) % 3
            p6, p3 = (c - 2) % 6, (c - 2) % 3
            _waitR(c6, c3)
            if c == 5:
                @pl.when(i < _NT - 1)
                def _():
                    _waitE(n6)
            else:
                _waitE(n6)
            if c < 2:
                @pl.when(i > 0)
                def _():
                    _waitS(p6, p3)
            else:
                _waitS(p6, p3)
            if c == 5:
                @pl.when(i < _NT - 1)
                def _():
                    _startR(n6, n3)
            else:
                _startR(n6, n3)
            if c < 4:
                _startE(base + (c + 2) * CHUNK, (c + 2) % 6)
            else:
                @pl.when(i < _NT - 1)
                def _():
                    _startE(base + (c + 2) * CHUNK, (c + 2) % 6)
            _compute(c6, c3)
            _startS(c6, c3)
        return _
    lax.fori_loop(0, _NT, _iter, None)

    _waitS((_NCK - 2) % 6, (_NCK - 2) % 3)
    _waitS((_NCK - 1) % 6, (_NCK - 1) % 3)

    plsc.subcore_barrier()
    pltpu.sync_copy(accs.at[pl.ds(abase, RPT)],
                    out_hbm.at[cid, pl.ds(abase, RPT)])


def _sc_edge(g16, ad, edges):
    mesh = plsc.VectorSubcoreMesh(core_axis_name="c", subcore_axis_name="s")
    f = pl.kernel(
        _edge_body,
        out_type=jax.ShapeDtypeStruct((2, NPAD, 16), _f32),
        mesh=mesh,
        compiler_params=pltpu.CompilerParams(needs_layout_passes=False,
                                             use_tc_tiling_on_sc=False),
        scratch_types=[
            pltpu.VMEM_SHARED((NPAD, 16), _f32),
            pltpu.VMEM_SHARED((NPAD,), _f32),
            [pltpu.VMEM((2, CHUNK), _i32) for _ in range(6)],
            [pltpu.VMEM((CHUNK, 16), _f32) for _ in range(3)],
            [pltpu.VMEM((CHUNK,), _f32) for _ in range(3)],
            [pltpu.SemaphoreType.DMA for _ in range(6)],
            [pltpu.SemaphoreType.DMA for _ in range(3)],
            [pltpu.SemaphoreType.DMA for _ in range(3)],
            [pltpu.SemaphoreType.DMA for _ in range(3)],
        ],
    )
    return f(g16, ad, edges)


NPW = NPAD // NW


PC = NPW // 2


def _pool_body(hfin_hbm, batch_hbm, psum_hbm, pmax_hbm,
               rowsb, batchb, sacc, macc, sem):
    cid = lax.axis_index("c")
    sid = lax.axis_index("s")
    wid = cid * 16 + sid
    iota = lax.iota(_i32, 16)
    zeros16 = jnp.zeros((16,), _f32)
    neg16 = jnp.full((16,), -3.4e38, _f32)

    def _init(i, _):
        idx = jnp.broadcast_to(i, (16,))
        plsc.store_scatter(sacc, [idx, iota], zeros16)
        plsc.store_scatter(macc, [idx, iota], neg16)
        return _
    lax.fori_loop(0, GP, _init, None)

    def _chunk(k, _):
        nb = wid * NPW + k * PC
        pltpu.sync_copy(hfin_hbm.at[pl.ds(nb, PC)], rowsb)
        pltpu.sync_copy(batch_hbm.at[pl.ds(nb, PC)], batchb)

        def _node(i, _):
            ridx = jnp.broadcast_to(i, (16,))
            bidx = plsc.load_gather(batchb, [ridx])
            row = plsc.load_gather(rowsb, [ridx, iota])
            s_old = plsc.load_gather(sacc, [bidx, iota])
            plsc.store_scatter(sacc, [bidx, iota], s_old + row)
            m_old = plsc.load_gather(macc, [bidx, iota])
            plsc.store_scatter(macc, [bidx, iota], jnp.maximum(m_old, row))
            return _
        lax.fori_loop(0, PC, _node, None)
        return _
    lax.fori_loop(0, NPW // PC, _chunk, None)

    pltpu.sync_copy(sacc, psum_hbm.at[wid])
    pltpu.sync_copy(macc, pmax_hbm.at[wid])


def _sc_pool(hfin, batch_p):
    mesh = plsc.VectorSubcoreMesh(core_axis_name="c", subcore_axis_name="s")
    f = pl.kernel(
        _pool_body,
        out_type=(jax.ShapeDtypeStruct((NW, GP, 16), _f32),
                  jax.ShapeDtypeStruct((NW, GP, 16), _f32)),
        mesh=mesh,
        compiler_params=pltpu.CompilerParams(needs_layout_passes=False,
                                             use_tc_tiling_on_sc=False),
        scratch_types=[
            pltpu.VMEM((PC, 16), _f32),
            pltpu.VMEM((PC,), _i32),
            pltpu.VMEM((GP, 16), _f32),
            pltpu.VMEM((GP, 16), _f32),
            pltpu.SemaphoreType.DMA,
        ],
    )
    return f(hfin, batch_p)


def kernel(x, edge_index, batch, Wg1, asrc1, adst1, bg1, Wf1, bf1,
           Wg2, asrc2, adst2, bg2, Wf2, bf2,
           Wg3, asrc3, adst3, bg3, Wf3, bf3,
           Wm1, bm1, Wm2, bm2):
    loops = jnp.arange(N, dtype=_i32)
    epad = jnp.full((2, EPAD - _E2), N, _i32)
    loops2 = jnp.stack([loops, loops])
    edges = jnp.concatenate([edge_index, loops2, epad], axis=1)
    x_p = jnp.pad(x, ((0, NPAD - N), (0, 0)))
    batch_p = jnp.pad(batch, (0, NPAD - N), constant_values=G)

    g16, ad = _tc_pre(x_p, Wg1, asrc1.reshape(NF, 1), adst1.reshape(NF, 1))
    acc = _sc_edge(g16, ad.reshape(NPAD), edges)
    g16, ad = _tc_mid(acc, bg1.reshape(1, NF), Wf1, bf1.reshape(1, NF),
                      Wg2, asrc2.reshape(NF, 1), adst2.reshape(NF, 1))
    acc = _sc_edge(g16, ad.reshape(NPAD), edges)
    g16, ad = _tc_mid(acc, bg2.reshape(1, NF), Wf2, bf2.reshape(1, NF),
                      Wg3, asrc3.reshape(NF, 1), adst3.reshape(NF, 1))
    acc = _sc_edge(g16, ad.reshape(NPAD), edges)
    hfin = _tc_post(acc, bg3.reshape(1, NF), Wf3, bf3.reshape(1, NF))
    psum, pmax = _sc_pool(hfin, batch_p)
    wm2_p = jnp.pad(Wm2, ((0, 7), (0, 0)))
    bm2_p = jnp.pad(bm2.reshape(1, 1), ((0, 0), (0, 7)))
    return _tc_head(psum, pmax, Wm1, bm1.reshape(1, 3 * NF), wm2_p, bm2_p)


# speedup vs baseline: 99.3405x; 1.2703x over previous
import functools

import jax
import jax.numpy as jnp
from jax import lax
from jax.experimental import pallas as pl
from jax.experimental.pallas import tpu as pltpu
from jax.experimental.pallas import tpu_sc as plsc

NF = 15
N = 50000
G = 512

NPAD = 50176
GP = 520
NW = 32
CHUNK = 480
RPT = NPAD // 16

_E2 = 1600000 + N
_NCK = -(-_E2 // (NW * CHUNK * 6)) * 6
EPW = _NCK * CHUNK
EPAD = NW * EPW
_NT = _NCK // 6

_f32 = jnp.float32
_i32 = jnp.int32
_PREC = jax.lax.Precision.HIGHEST


GR = NPAD * 16 // 128


def _dot(a, b):
    return jax.lax.dot_general(a, b, (((1,), (0,)), ((), ())),
                               preferred_element_type=_f32,
                               precision=_PREC)


def _tc_pre_body(xg_ref, a1_ref, d1_ref, g_ref, ad_ref):
    xg = xg_ref[...]
    g_ref[...] = _dot(xg, a1_ref[...])
    ad_ref[...] = _dot(xg, d1_ref[...])


def _tc_pre(xg, a1, d1):
    return pl.pallas_call(
        _tc_pre_body,
        out_shape=(jax.ShapeDtypeStruct((GR, 128), _f32),
                   jax.ShapeDtypeStruct((GR, 128), _f32)),
    )(xg, a1, d1)


def _gat_grouped(acc_ref, sb_ref, bg_ref):
    s = acc_ref[0] + acc_ref[1]
    rec = 1.0 / jnp.maximum(s, 1e-30)
    den = _dot(rec, sb_ref[...])
    return s * den + bg_ref[...]


def _tc_mid_body(acc_ref, sb_ref, bg_ref, wf_ref, bf_ref, m_ref, m2_ref,
                 g_ref, ad_ref):
    gat = _gat_grouped(acc_ref, sb_ref, bg_ref)
    hf = jax.nn.relu(_dot(gat, wf_ref[...]) + bf_ref[...])
    g_ref[...] = _dot(hf, m_ref[...])
    ad_ref[...] = _dot(hf, m2_ref[...])


def _tc_mid(acc, sb, bg, wf, bf, m, m2):
    return pl.pallas_call(
        _tc_mid_body,
        out_shape=(jax.ShapeDtypeStruct((GR, 128), _f32),
                   jax.ShapeDtypeStruct((GR, 128), _f32)),
    )(acc, sb, bg, wf, bf, m, m2)


def _tc_post_body(acc_ref, sb_ref, bg_ref, wf_ref, bf_ref, h_ref):
    gat = _gat_grouped(acc_ref, sb_ref, bg_ref)
    h_ref[...] = _dot(gat, wf_ref[...]) + bf_ref[...]


def _tc_post(acc, sb, bg, wf, bf):
    return pl.pallas_call(
        _tc_post_body,
        out_shape=jax.ShapeDtypeStruct((GR, 128), _f32),
    )(acc, sb, bg, wf, bf)


def _lanes15(col):
    return jax.lax.dot_general(col, jnp.ones((1, NF), _f32),
                               (((1,), (0,)), ((), ())),
                               preferred_element_type=_f32,
                               precision=_PREC)


def _tc_head_body(psum_ref, pmax_ref, wm1_ref, bm1_ref, wm2_ref, bm2_ref,
                  out_ref):
    ssum = jnp.sum(psum_ref[...], axis=0)[:G]
    smaxr = jnp.max(pmax_ref[...], axis=0)[:G]
    cnt = ssum[:, 15:16]
    smax = jnp.where(_lanes15(cnt) > 0, smaxr[:, :15], 0.0)
    smean = ssum[:, :15] * _lanes15(1.0 / jnp.maximum(cnt, 1.0))
    z = jnp.concatenate([ssum[:, :15], smax, smean], axis=1)
    z = jax.nn.relu(
        jax.lax.dot_general(z, wm1_ref[...], (((1,), (1,)), ((), ())),
                            preferred_element_type=_f32, precision=_PREC)
        + bm1_ref[...])
    z = jax.lax.dot_general(z, wm2_ref[...], (((1,), (1,)), ((), ())),
                            preferred_element_type=_f32, precision=_PREC) \
        + bm2_ref[...]
    out_ref[...] = jax.nn.sigmoid(z)[:, :1]


def _tc_head(psum, pmax, wm1, bm1, wm2, bm2):
    return pl.pallas_call(
        _tc_head_body,
        out_shape=jax.ShapeDtypeStruct((G, 1), _f32),
    )(psum, pmax, wm1, bm1, wm2, bm2)


def _edge_body(g16_hbm, ad_hbm, edges_hbm, out_hbm, accs, ads,
               ebs, rws, ads_ch, ses, srs, sas, sss):
    cid = lax.axis_index("c")
    sid = lax.axis_index("s")
    wid = cid * 16 + sid
    iota = lax.iota(_i32, 16)
    zeros16 = jnp.zeros((16,), _f32)
    rows0 = rws[0]

    def _zrow(i, _):
        plsc.store_scatter(rows0, [jnp.broadcast_to(i, (16,)), iota],
                           zeros16)
        return _
    lax.fori_loop(0, CHUNK, _zrow, None)
    abase = sid * RPT
    for j in range(RPT // CHUNK):
        pltpu.sync_copy(rows0, accs.at[pl.ds(abase + j * CHUNK, CHUNK)])
    rem = RPT % CHUNK
    if rem:
        pltpu.sync_copy(rows0.at[pl.ds(0, rem)],
                        accs.at[pl.ds(abase + (RPT // CHUNK) * CHUNK, rem)])

    pltpu.sync_copy(ad_hbm.at[pl.ds(sid * RPT, RPT)],
                    ads.at[pl.ds(sid * RPT, RPT)])
    plsc.subcore_barrier()

    ebase = wid * EPW

    def _startE(off, c6):
        pltpu.async_copy(edges_hbm.at[:, pl.ds(off, CHUNK)], ebs[c6],
                         ses[c6])

    def _waitE(c6):
        pltpu.make_async_copy(edges_hbm.at[:, pl.ds(0, CHUNK)], ebs[c6],
                              ses[c6]).wait()

    def _startR(c6, c3):
        pltpu.async_copy(g16_hbm.at[ebs[c6].at[0]], rws[c3], srs[c3])
        pltpu.async_copy(ads.at[ebs[c6].at[1]], ads_ch[c3], sas[c3])

    def _waitR(c6, c3):
        pltpu.make_async_copy(g16_hbm.at[ebs[c6].at[0]], rws[c3],
                              srs[c3]).wait()
        pltpu.make_async_copy(ads.at[ebs[c6].at[1]], ads_ch[c3],
                              sas[c3]).wait()

    def _startS(c6, c3):
        pltpu.async_copy(rws[c3], accs.at[ebs[c6].at[1]], sss[c3],
                         add=True)

    def _waitS(c6, c3):
        pltpu.make_async_copy(rws[c3], accs.at[ebs[c6].at[1]],
                              sss[c3]).wait()

    def _compute(c6, c3):
        rows, adch = rws[c3], ads_ch[c3]
        lane15 = jnp.broadcast_to(15, (16,))

        def _grp(j, _):
            rid = iota + j * 16
            adv = plsc.load_gather(adch, [rid])
            asv = plsc.load_gather(rows, [rid, lane15])
            e = asv + adv
            w = jnp.exp(jnp.where(e >= 0, e, 0.2 * e))
            plsc.store_scatter(rows, [rid, lane15], w)
            for f in range(15):
                lane = jnp.broadcast_to(f, (16,))
                col = plsc.load_gather(rows, [rid, lane])
                plsc.store_scatter(rows, [rid, lane], col * w)
            return _
        lax.fori_loop(0, CHUNK // 16, _grp, None)

    pltpu.sync_copy(edges_hbm.at[:, pl.ds(ebase, CHUNK)], ebs[0])
    _startR(0, 0)
    _startE(ebase + CHUNK, 1)

    def _iter(i, _):
        base = ebase + 6 * i * CHUNK
        for c in range(6):
            c6, c3 = c, c % 3
            n6, n3 = (c + 1) % 6, (c + 1) % 3
            p6, p3 = (c - 2) % 6, (c - 2) % 3
            _waitR(c6, c3)
            if c == 5:
                @pl.when(i < _NT - 1)
                def _():
                    _waitE(n6)
            else:
                _waitE(n6)
            if c < 2:
                @pl.when(i > 0)
                def _():
                    _waitS(p6, p3)
            else:
                _waitS(p6, p3)
            if c == 5:
                @pl.when(i < _NT - 1)
                def _():
                    _startR(n6, n3)
            else:
                _startR(n6, n3)
            if c < 4:
                _startE(base + (c + 2) * CHUNK, (c + 2) % 6)
            else:
                @pl.when(i < _NT - 1)
                def _():
                    _startE(base + (c + 2) * CHUNK, (c + 2) % 6)
            _compute(c6, c3)
            _startS(c6, c3)
        return _
    lax.fori_loop(0, _NT, _iter, None)

    _waitS((_NCK - 2) % 6, (_NCK - 2) % 3)
    _waitS((_NCK - 1) % 6, (_NCK - 1) % 3)

    plsc.subcore_barrier()
    pltpu.sync_copy(accs.at[pl.ds(abase, RPT)],
                    out_hbm.at[cid, pl.ds(abase, RPT)])


def _sc_edge(g16, ad, edges):
    mesh = plsc.VectorSubcoreMesh(core_axis_name="c", subcore_axis_name="s")
    f = pl.kernel(
        _edge_body,
        out_type=jax.ShapeDtypeStruct((2, NPAD, 16), _f32),
        mesh=mesh,
        compiler_params=pltpu.CompilerParams(needs_layout_passes=False,
                                             use_tc_tiling_on_sc=False),
        scratch_types=[
            pltpu.VMEM_SHARED((NPAD, 16), _f32),
            pltpu.VMEM_SHARED((NPAD,), _f32),
            [pltpu.VMEM((2, CHUNK), _i32) for _ in range(6)],
            [pltpu.VMEM((CHUNK, 16), _f32) for _ in range(3)],
            [pltpu.VMEM((CHUNK,), _f32) for _ in range(3)],
            [pltpu.SemaphoreType.DMA for _ in range(6)],
            [pltpu.SemaphoreType.DMA for _ in range(3)],
            [pltpu.SemaphoreType.DMA for _ in range(3)],
            [pltpu.SemaphoreType.DMA for _ in range(3)],
        ],
    )
    return f(g16, ad, edges)


NPW = NPAD // NW


PC = NPW // 2


def _pool_body(hfin_hbm, batch_hbm, psum_hbm, pmax_hbm,
               rowsb, batchb, sacc, macc, sem):
    cid = lax.axis_index("c")
    sid = lax.axis_index("s")
    wid = cid * 16 + sid
    iota = lax.iota(_i32, 16)
    zeros16 = jnp.zeros((16,), _f32)
    neg16 = jnp.full((16,), -3.4e38, _f32)

    def _init(i, _):
        idx = jnp.broadcast_to(i, (16,))
        plsc.store_scatter(sacc, [idx, iota], zeros16)
        plsc.store_scatter(macc, [idx, iota], neg16)
        return _
    lax.fori_loop(0, GP, _init, None)

    def _chunk(k, _):
        nb = wid * NPW + k * PC
        pltpu.sync_copy(hfin_hbm.at[pl.ds(nb, PC)], rowsb)
        pltpu.sync_copy(batch_hbm.at[pl.ds(nb, PC)], batchb)

        def _node(i, _):
            ridx = jnp.broadcast_to(i, (16,))
            bidx = plsc.load_gather(batchb, [ridx])
            row = plsc.load_gather(rowsb, [ridx, iota])
            s_old = plsc.load_gather(sacc, [bidx, iota])
            plsc.store_scatter(sacc, [bidx, iota], s_old + row)
            m_old = plsc.load_gather(macc, [bidx, iota])
            plsc.store_scatter(macc, [bidx, iota], jnp.maximum(m_old, row))
            return _
        lax.fori_loop(0, PC, _node, None)
        return _
    lax.fori_loop(0, NPW // PC, _chunk, None)

    pltpu.sync_copy(sacc, psum_hbm.at[wid])
    pltpu.sync_copy(macc, pmax_hbm.at[wid])


def _sc_pool(hfin, batch_p):
    mesh = plsc.VectorSubcoreMesh(core_axis_name="c", subcore_axis_name="s")
    f = pl.kernel(
        _pool_body,
        out_type=(jax.ShapeDtypeStruct((NW, GP, 16), _f32),
                  jax.ShapeDtypeStruct((NW, GP, 16), _f32)),
        mesh=mesh,
        compiler_params=pltpu.CompilerParams(needs_layout_passes=False,
                                             use_tc_tiling_on_sc=False),
        scratch_types=[
            pltpu.VMEM((PC, 16), _f32),
            pltpu.VMEM((PC,), _i32),
            pltpu.VMEM((GP, 16), _f32),
            pltpu.VMEM((GP, 16), _f32),
            pltpu.SemaphoreType.DMA,
        ],
    )
    return f(hfin, batch_p)


def kernel(x, edge_index, batch, Wg1, asrc1, adst1, bg1, Wf1, bf1,
           Wg2, asrc2, adst2, bg2, Wf2, bf2,
           Wg3, asrc3, adst3, bg3, Wf3, bf3,
           Wm1, bm1, Wm2, bm2):
    loops = jnp.arange(N, dtype=_i32)
    epad = jnp.full((2, EPAD - _E2), N, _i32)
    loops2 = jnp.stack([loops, loops])
    edges = jnp.concatenate([edge_index, loops2, epad], axis=1)
    x_p = jnp.pad(x, ((0, NPAD - N), (0, 1)))
    xg = x_p.reshape(GR, 128)
    batch_p = jnp.pad(batch, (0, NPAD - N), constant_values=G)

    eye8 = jnp.eye(8, dtype=_f32)

    def bd(m16):
        return jnp.kron(eye8, m16)

    def a_mat(wg, asrc):
        m = jnp.zeros((16, 16), _f32)
        m = m.at[:NF, :NF].set(wg.T)
        return bd(m.at[:NF, NF].set(wg.T @ asrc))

    def d_mat(wg, adst):
        m = jnp.zeros((16, 16), _f32)
        return bd(m.at[:NF, 0].set(wg.T @ adst))

    def w_mat(wf):
        m = jnp.zeros((16, 16), _f32)
        return bd(m.at[:NF, :NF].set(wf.T))

    def tile8(v16):
        return jnp.tile(v16, 8).reshape(1, 128)

    sb = bd(jnp.zeros((16, 16), _f32).at[15, :].set(1.0))
    z16 = jnp.zeros((1,), _f32)

    def ad_col(adg):
        return adg.reshape(NPAD, 16)[:, 0]

    g16, adg = _tc_pre(xg, a_mat(Wg1, asrc1), d_mat(Wg1, adst1))
    acc = _sc_edge(g16.reshape(NPAD, 16), ad_col(adg), edges)
    g16, adg = _tc_mid(acc.reshape(2, GR, 128), sb,
                       tile8(jnp.concatenate([bg1, z16])),
                       w_mat(Wf1), tile8(jnp.concatenate([bf1, z16])),
                       a_mat(Wg2, asrc2), d_mat(Wg2, adst2))
    acc = _sc_edge(g16.reshape(NPAD, 16), ad_col(adg), edges)
    g16, adg = _tc_mid(acc.reshape(2, GR, 128), sb,
                       tile8(jnp.concatenate([bg2, z16])),
                       w_mat(Wf2), tile8(jnp.concatenate([bf2, z16])),
                       a_mat(Wg3, asrc3), d_mat(Wg3, adst3))
    acc = _sc_edge(g16.reshape(NPAD, 16), ad_col(adg), edges)
    hfin = _tc_post(acc.reshape(2, GR, 128), sb,
                    tile8(jnp.concatenate([bg3, z16])),
                    w_mat(Wf3),
                    tile8(jnp.concatenate([bf3, jnp.ones((1,), _f32)])))
    psum, pmax = _sc_pool(hfin.reshape(NPAD, 16), batch_p)
    wm2_p = jnp.pad(Wm2, ((0, 7), (0, 0)))
    bm2_p = jnp.pad(bm2.reshape(1, 1), ((0, 0), (0, 7)))
    return _tc_head(psum, pmax, Wm1, bm1.reshape(1, 3 * NF), wm2_p, bm2_p)
